# baseline (speedup 1.0000x reference)
import functools

import jax
import jax.numpy as jnp
from jax import lax
from jax.experimental import pallas as pl
from jax.experimental.pallas import tpu as pltpu
from jax.experimental.pallas import tpu_sc as plsc

NUM_CORES = 2
NUM_SUBCORES = 16
LANES = 16
NUM_WORKERS = NUM_CORES * NUM_SUBCORES
CHUNK = 32


def kernel(x, pe_weight):
    batch, seq_len, d = x.shape
    seq_per_w = seq_len // NUM_WORKERS
    steps = seq_per_w // CHUNK
    assert seq_len % NUM_WORKERS == 0 and seq_per_w % CHUNK == 0
    words = CHUNK * d

    xf = x.reshape(batch * seq_len * d)
    pef = pe_weight.reshape(pe_weight.shape[0] * d)

    mesh = plsc.VectorSubcoreMesh(
        core_axis_name="c", subcore_axis_name="s",
        num_cores=NUM_CORES, num_subcores=NUM_SUBCORES,
    )

    @functools.partial(
        pl.kernel,
        out_type=jax.ShapeDtypeStruct((batch * seq_len * d,), x.dtype),
        mesh=mesh,
        scratch_types=[
            pltpu.VMEM((words,), jnp.float32),
            pltpu.VMEM((words,), jnp.float32),
        ],
    )
    def sc_add(x_hbm, pe_hbm, out_hbm, pe_v, x_v):
        wid = lax.axis_index("s") * NUM_CORES + lax.axis_index("c")
        s_base = wid * seq_per_w

        @pl.loop(0, steps)
        def step(i):
            s0 = (s_base + i * CHUNK) * d
            pltpu.sync_copy(pe_hbm.at[pl.ds(s0, words)], pe_v)
            for b in range(batch):
                x0 = b * seq_len * d + s0
                pltpu.sync_copy(x_hbm.at[pl.ds(x0, words)], x_v)

                @plsc.parallel_loop(0, words, LANES, unroll=8)
                def add_body(o):
                    plsc.addupdate(x_v.at[pl.ds(o, LANES)], pe_v[pl.ds(o, LANES)])

                pltpu.sync_copy(x_v, out_hbm.at[pl.ds(x0, words)])

    out = sc_add(xf, pef)
    return out.reshape(batch, seq_len, d)


# speedup vs baseline: 1.2405x; 1.2405x over previous
import functools

import jax
import jax.numpy as jnp
from jax import lax
from jax.experimental import pallas as pl
from jax.experimental.pallas import tpu as pltpu
from jax.experimental.pallas import tpu_sc as plsc

NUM_CORES = 2
NUM_SUBCORES = 16
LANES = 16
NUM_WORKERS = NUM_CORES * NUM_SUBCORES
CHUNK = 32
SEQ_BLOCK = 512
SPLIT = 3


def _tc_body(x_ref, pe_ref, o_ref):
    o_ref[...] = x_ref[...] + pe_ref[...][None, :, :]


def _tc_add(x, pe, d):
    batch, seq_len = x.shape[0], x.shape[1]
    num_seq_blocks = seq_len // SEQ_BLOCK
    return pl.pallas_call(
        _tc_body,
        grid=(num_seq_blocks, batch),
        in_specs=[
            pl.BlockSpec((1, SEQ_BLOCK, d), lambda i, j: (j, i, 0)),
            pl.BlockSpec((SEQ_BLOCK, d), lambda i, j: (i, 0)),
        ],
        out_specs=pl.BlockSpec((1, SEQ_BLOCK, d), lambda i, j: (j, i, 0)),
        out_shape=jax.ShapeDtypeStruct(x.shape, x.dtype),
        compiler_params=pltpu.CompilerParams(
            dimension_semantics=("arbitrary", "arbitrary"),
        ),
    )(x, pe)


def _sc_add(xf, pef, batch, seq_len, d):
    seq_per_w = seq_len // NUM_WORKERS
    steps = seq_per_w // CHUNK
    assert seq_len % NUM_WORKERS == 0 and seq_per_w % CHUNK == 0
    words = CHUNK * d

    mesh = plsc.VectorSubcoreMesh(
        core_axis_name="c", subcore_axis_name="s",
        num_cores=NUM_CORES, num_subcores=NUM_SUBCORES,
    )

    @functools.partial(
        pl.kernel,
        out_type=jax.ShapeDtypeStruct((batch * seq_len * d,), xf.dtype),
        mesh=mesh,
        scratch_types=[
            pltpu.VMEM((words,), jnp.float32),
            pltpu.VMEM((words,), jnp.float32),
        ],
    )
    def sc_add(x_hbm, pe_hbm, out_hbm, pe_v, x_v):
        wid = lax.axis_index("s") * NUM_CORES + lax.axis_index("c")
        s_base = wid * seq_per_w

        @pl.loop(0, steps)
        def step(i):
            s0 = (s_base + i * CHUNK) * d
            pltpu.sync_copy(pe_hbm.at[pl.ds(s0, words)], pe_v)
            for b in range(batch):
                x0 = b * seq_len * d + s0
                pltpu.sync_copy(x_hbm.at[pl.ds(x0, words)], x_v)

                @plsc.parallel_loop(0, words, LANES, unroll=8)
                def add_body(o):
                    plsc.addupdate(x_v.at[pl.ds(o, LANES)], pe_v[pl.ds(o, LANES)])

                pltpu.sync_copy(x_v, out_hbm.at[pl.ds(x0, words)])

    return sc_add(xf, pef)


def kernel(x, pe_weight):
    batch, seq_len, d = x.shape
    pe = pe_weight[:seq_len]

    tc_out = _tc_add(x[:SPLIT], pe, d)

    sc_batch = batch - SPLIT
    xf = x[SPLIT:].reshape(sc_batch * seq_len * d)
    pef = pe.reshape(seq_len * d)
    sc_out = _sc_add(xf, pef, sc_batch, seq_len, d).reshape(sc_batch, seq_len, d)

    return jnp.concatenate([tc_out, sc_out], axis=0)


# speedup vs baseline: 4.9723x; 4.0083x over previous
import jax
import jax.numpy as jnp
from jax.experimental import pallas as pl
from jax.experimental.pallas import tpu as pltpu

SEQ_BLOCK = 1024


def _add_body(x_ref, pe_ref, o_ref):
    o_ref[...] = x_ref[...] + pe_ref[...][None, :, :]


def kernel(x, pe_weight):
    batch, seq_len, d_model = x.shape
    pe = pe_weight[:seq_len]
    num_seq_blocks = seq_len // SEQ_BLOCK

    grid = (num_seq_blocks, batch)
    return pl.pallas_call(
        _add_body,
        grid=grid,
        in_specs=[
            pl.BlockSpec((1, SEQ_BLOCK, d_model), lambda i, j: (j, i, 0)),
            pl.BlockSpec((SEQ_BLOCK, d_model), lambda i, j: (i, 0)),
        ],
        out_specs=pl.BlockSpec((1, SEQ_BLOCK, d_model), lambda i, j: (j, i, 0)),
        out_shape=jax.ShapeDtypeStruct(x.shape, x.dtype),
        compiler_params=pltpu.CompilerParams(
            dimension_semantics=("arbitrary", "arbitrary"),
        ),
    )(x, pe)


# speedup vs baseline: 5.1888x; 1.0435x over previous
import jax
import jax.numpy as jnp
from jax.experimental import pallas as pl
from jax.experimental.pallas import tpu as pltpu

SEQ_BLOCK = 2048


def _add_body(x_ref, pe_ref, o_ref):
    o_ref[...] = x_ref[...] + pe_ref[...][None, :, :]


def kernel(x, pe_weight):
    batch, seq_len, d_model = x.shape
    pe = pe_weight[:seq_len]
    num_seq_blocks = seq_len // SEQ_BLOCK

    grid = (num_seq_blocks, batch)
    return pl.pallas_call(
        _add_body,
        grid=grid,
        in_specs=[
            pl.BlockSpec((1, SEQ_BLOCK, d_model), lambda i, j: (j, i, 0)),
            pl.BlockSpec((SEQ_BLOCK, d_model), lambda i, j: (i, 0)),
        ],
        out_specs=pl.BlockSpec((1, SEQ_BLOCK, d_model), lambda i, j: (j, i, 0)),
        out_shape=jax.ShapeDtypeStruct(x.shape, x.dtype),
        compiler_params=pltpu.CompilerParams(
            dimension_semantics=("arbitrary", "arbitrary"),
        ),
    )(x, pe)
